# Initial kernel scaffold; baseline (speedup 1.0000x reference)
import jax
import jax.numpy as jnp
from jax.experimental import pallas as pl


def kernel(x, pos, edge_index, edge_attr, params):
    raise NotImplementedError("write your pallas kernel here")


# speedup vs baseline: 2.3303x; 2.3303x over previous
import functools

import jax
import jax.numpy as jnp
from jax import lax
from jax.experimental import pallas as pl
from jax.experimental.pallas import tpu as pltpu
from jax.experimental.pallas import tpu_sc as plsc

N = 10000
E = 160000
D = 128
DE = 16
DEH = 64
DM = 32
NIT = 3

NC, NS = 2, 16
NW = NC * NS
EPAD = 163840
EPW = EPAD // NW
CH = 128
NCHUNK = EPW // CH
PW = 16
VW = DM + PW
NPT = N // NS
EBLK = 2048
EPS = 1e-5


def _make_gather(specs):
    n = len(specs)
    mesh = plsc.VectorSubcoreMesh(core_axis_name="c", subcore_axis_name="s")
    out_type = [jax.ShapeDtypeStruct((EPAD, w), jnp.float32) for w, _ in specs]
    scratch = [pltpu.VMEM((CH,), jnp.int32), pltpu.VMEM((CH,), jnp.int32)]
    scratch += [pltpu.VMEM((CH, w), jnp.float32) for w, _ in specs]
    scratch += [pltpu.SemaphoreType.DMA for _ in specs]

    @functools.partial(pl.kernel, out_type=out_type, mesh=mesh,
                       scratch_types=scratch,
                       compiler_params=pltpu.CompilerParams(
                           use_tc_tiling_on_sc=False))
    def k(*refs):
        tables = refs[:n]
        dstp, srcp = refs[n], refs[n + 1]
        outs = refs[n + 2:2 * n + 2]
        idxd, idxs = refs[2 * n + 2], refs[2 * n + 3]
        rows = refs[2 * n + 4:3 * n + 4]
        sems = refs[3 * n + 4:]
        wid = lax.axis_index("s") * NC + lax.axis_index("c")
        base = pl.multiple_of(wid * EPW, EPW)

        def body(j, carry):
            off = pl.multiple_of(base + j * CH, CH)
            pltpu.sync_copy(dstp.at[pl.ds(off, CH)], idxd)
            pltpu.sync_copy(srcp.at[pl.ds(off, CH)], idxs)
            cps = []
            for t, (_, use_src) in enumerate(specs):
                idx = idxs if use_src else idxd
                cps.append(pltpu.async_copy(tables[t].at[idx], rows[t], sems[t]))
            for cp in cps:
                cp.wait()
            for t in range(n):
                pltpu.sync_copy(rows[t], outs[t].at[pl.ds(off, CH)])
            return carry

        lax.fori_loop(0, NCHUNK, body, 0)

    return k


def _make_scatter():
    mesh = plsc.VectorSubcoreMesh(core_axis_name="c", subcore_axis_name="s")

    @functools.partial(
        pl.kernel,
        out_type=jax.ShapeDtypeStruct((NC, N, VW), jnp.float32),
        mesh=mesh,
        scratch_types=[
            pltpu.VMEM((NCHUNK, CH), jnp.int32),
            pltpu.VMEM((CH, VW), jnp.float32),
            pltpu.VMEM_SHARED((N, VW), jnp.float32),
        ],
        compiler_params=pltpu.CompilerParams(use_tc_tiling_on_sc=False),
    )
    def k(vals, dst2d, zrows, out, idxv, valv, acc):
        c = lax.axis_index("c")
        s = lax.axis_index("s")
        wid = s * NC + c

        @pl.when(s == 0)
        def _zero():
            pltpu.sync_copy(zrows, acc)

        plsc.subcore_barrier()
        pltpu.sync_copy(dst2d.at[pl.ds(wid * NCHUNK, NCHUNK)], idxv)

        def body(j, carry):
            off = pl.multiple_of(wid * EPW + j * CH, CH)
            pltpu.sync_copy(vals.at[pl.ds(off, CH)], valv)
            pltpu.sync_copy(valv, acc.at[idxv.at[j]], add=True)
            return carry

        lax.fori_loop(0, NCHUNK, body, 0)
        plsc.subcore_barrier()
        r0 = pl.multiple_of(s * NPT, NPT)
        pltpu.sync_copy(acc.at[pl.ds(r0, NPT)], out.at[c, pl.ds(r0, NPT)])

    return k


_gather_pos = _make_gather(((PW, False), (PW, True)))
_gather_feat = _make_gather(((DEH, False), (DEH, True)))
_gather_feat_pos = _make_gather(((DEH, False), (DEH, True), (PW, False), (PW, True)))
_scatter = _make_scatter()


def _gelu(v):
    return jax.nn.gelu(v, approximate=True)


def _ln(v, g, b):
    m = jnp.mean(v, axis=-1, keepdims=True)
    c = v - m
    s = jnp.mean(c * c, axis=-1, keepdims=True)
    return c / jnp.sqrt(s + EPS) * g + b


def _bn(v, g, b):
    m = jnp.mean(v, axis=0, keepdims=True)
    c = v - m
    s = jnp.mean(c * c, axis=0, keepdims=True)
    return c / jnp.sqrt(s + EPS) * g + b


def _dot(a, b):
    return jnp.dot(a, b, preferred_element_type=jnp.float32)


def _prep_node_body(x_ref, winj_ref, binj_ref, gi_ref, bi_ref, go_ref, bo_ref,
                    xinj_ref):
    v = _ln(x_ref[...], gi_ref[0:1], bi_ref[0:1])
    v = _dot(v, winj_ref[...]) + binj_ref[0:1]
    xinj_ref[...] = _ln(v, go_ref[0:1], bo_ref[0:1])


def _prep_edge_body(pa_ref, pb_ref, ea_ref, wea0_ref, be10_ref, wd20_ref,
                    wea1_ref, be11_ref, eaw0_ref, eaw1_ref, rel0_ref):
    rel = pb_ref[...] - pa_ref[...]
    d2 = jnp.sum(rel * rel, axis=1, keepdims=True)
    eaw0_ref[...] = _dot(ea_ref[...], wea0_ref[...]) + be10_ref[0:1] \
        + d2 * wd20_ref[0:1]
    eaw1_ref[...] = _dot(ea_ref[...], wea1_ref[...]) + be11_ref[0:1]
    rel0_ref[...] = rel


def _edge_tail(mpre, rel, we2_ref, be2_ref, wp_ref, vals_ref):
    m = _gelu(mpre)
    m2 = _gelu(_dot(m, we2_ref[...]) + be2_ref[0:1])
    w = jnp.sum(m2 * wp_ref[0:1], axis=1, keepdims=True)
    rows = pl.program_id(0) * EBLK + lax.broadcasted_iota(jnp.int32, (EBLK, 1), 0)
    mask = (rows < E).astype(jnp.float32)
    vals_ref[...] = jnp.concatenate([m2, rel * w], axis=1) * mask


def _edge0_first_body(eaw_ref, rel_ref, we2_ref, be2_ref, wp_ref, vals_ref):
    _edge_tail(eaw_ref[...], rel_ref[...], we2_ref, be2_ref, wp_ref, vals_ref)


def _edge0_body(a_ref, b_ref, eaw_ref, rel_ref, we2_ref, be2_ref, wp_ref,
                vals_ref):
    _edge_tail(a_ref[...] + b_ref[...] + eaw_ref[...], rel_ref[...],
               we2_ref, be2_ref, wp_ref, vals_ref)


def _edge1_body(a_ref, b_ref, pa_ref, pb_ref, eaw_ref, wd2_ref, we2_ref,
                be2_ref, wp_ref, vals_ref):
    rel = pb_ref[...] - pa_ref[...]
    d2 = jnp.sum(rel * rel, axis=1, keepdims=True)
    mpre = a_ref[...] + b_ref[...] + eaw_ref[...] + d2 * wd2_ref[0:1]
    _edge_tail(mpre, rel, we2_ref, be2_ref, wp_ref, vals_ref)


def _node0_body(z_ref, accs_ref, xinj_ref, q_ref, wh1h_ref, wh1a_ref, bh1_ref,
                wh2_ref, bh2_ref, g3_ref, b3_ref, pg1_ref, pb1_ref,
                wdst1_ref, wsrc1_ref,
                z0_ref, z1g_ref, p0_ref, ta1_ref, tb1_ref, p1_ref):
    acc = accs_ref[0] + accs_ref[1]
    agg = acc[:, :DM]
    p0 = acc[:, DM:]
    hin = _dot(z_ref[...], wh1h_ref[...]) + _dot(agg, wh1a_ref[...]) + bh1_ref[0:1]
    z0 = _dot(_gelu(hin), wh2_ref[...]) + bh2_ref[0:1]
    z1g = _gelu(_ln(z0 + xinj_ref[...], g3_ref[0:1], b3_ref[0:1]))
    p1 = _bn(p0 + q_ref[...], pg1_ref[0:1], pb1_ref[0:1])
    z0_ref[...] = z0
    z1g_ref[...] = z1g
    p0_ref[...] = p0
    ta1_ref[...] = _dot(z1g, wdst1_ref[...])
    tb1_ref[...] = _dot(z1g, wsrc1_ref[...])
    p1_ref[...] = p1


def _node1_body(z0_ref, z1g_ref, accs_ref, p0_ref, wh1h_ref, wh1a_ref,
                bh1_ref, wh2_ref, bh2_ref, g4_ref, b4_ref, pg2_ref, pb2_ref,
                wdst0_ref, wsrc0_ref,
                z_ref, q_out_ref, ta0_ref, tb0_ref):
    acc = accs_ref[0] + accs_ref[1]
    agg = acc[:, :DM]
    p2 = acc[:, DM:]
    hin = _dot(z1g_ref[...], wh1h_ref[...]) + _dot(agg, wh1a_ref[...]) + bh1_ref[0:1]
    z2 = _dot(_gelu(hin), wh2_ref[...]) + bh2_ref[0:1]
    zn = _ln(_gelu(z0_ref[...] + z2), g4_ref[0:1], b4_ref[0:1])
    qn = _bn(p0_ref[...] + p2, pg2_ref[0:1], pb2_ref[0:1])
    z_ref[...] = zn
    q_out_ref[...] = qn
    ta0_ref[...] = _dot(zn, wdst0_ref[...])
    tb0_ref[...] = _dot(zn, wsrc0_ref[...])


_TC_PARAMS = pltpu.CompilerParams(vmem_limit_bytes=100 * 1024 * 1024)


def _full(shape):
    return pl.BlockSpec(shape, lambda i: (0, 0))


def _eblk(width):
    return pl.BlockSpec((EBLK, width), lambda i: (i, 0))


_EGRID = EPAD // EBLK


def _prep_edge_call(pa, pb, ea, wea0, be10, wd20, wea1, be11):
    return pl.pallas_call(
        _prep_edge_body,
        grid=(_EGRID,),
        in_specs=[_eblk(PW), _eblk(PW), _eblk(DE), _full((DE, DEH)),
                  _full((8, DEH)), _full((8, DEH)), _full((DE, DEH)),
                  _full((8, DEH))],
        out_specs=[_eblk(DEH), _eblk(DEH), _eblk(PW)],
        out_shape=[jax.ShapeDtypeStruct((EPAD, DEH), jnp.float32),
                   jax.ShapeDtypeStruct((EPAD, DEH), jnp.float32),
                   jax.ShapeDtypeStruct((EPAD, PW), jnp.float32)],
    )(pa, pb, ea, wea0, be10, wd20, wea1, be11)


def _edge0_first_call(eaw, rel0, we2, be2, wp):
    return pl.pallas_call(
        _edge0_first_body,
        grid=(_EGRID,),
        in_specs=[_eblk(DEH), _eblk(PW), _full((DEH, DM)), _full((8, DM)),
                  _full((8, DM))],
        out_specs=_eblk(VW),
        out_shape=jax.ShapeDtypeStruct((EPAD, VW), jnp.float32),
    )(eaw, rel0, we2, be2, wp)


def _edge0_call(a, b, eaw, rel0, we2, be2, wp):
    return pl.pallas_call(
        _edge0_body,
        grid=(_EGRID,),
        in_specs=[_eblk(DEH), _eblk(DEH), _eblk(DEH), _eblk(PW),
                  _full((DEH, DM)), _full((8, DM)), _full((8, DM))],
        out_specs=_eblk(VW),
        out_shape=jax.ShapeDtypeStruct((EPAD, VW), jnp.float32),
    )(a, b, eaw, rel0, we2, be2, wp)


def _edge1_call(a, b, pa, pb, eaw, wd2, we2, be2, wp):
    return pl.pallas_call(
        _edge1_body,
        grid=(_EGRID,),
        in_specs=[_eblk(DEH), _eblk(DEH), _eblk(PW), _eblk(PW), _eblk(DEH),
                  _full((8, DEH)), _full((DEH, DM)), _full((8, DM)),
                  _full((8, DM))],
        out_specs=_eblk(VW),
        out_shape=jax.ShapeDtypeStruct((EPAD, VW), jnp.float32),
    )(a, b, pa, pb, eaw, wd2, we2, be2, wp)


def _r8(v):
    return jnp.broadcast_to(v.astype(jnp.float32)[None, :], (8, v.shape[0]))


def _pad16(v):
    return jnp.concatenate([v, jnp.zeros((PW - v.shape[0],), v.dtype)])


def kernel(x, pos, edge_index, edge_attr, params):
    src = edge_index[0].astype(jnp.int32)
    dst = edge_index[1].astype(jnp.int32)
    zpad = jnp.zeros((EPAD - E,), jnp.int32)
    dstp = jnp.concatenate([dst, zpad])
    srcp = jnp.concatenate([src, zpad])
    dst2d = dstp.reshape(EPAD // CH, CH)
    eap = jnp.concatenate(
        [edge_attr, jnp.zeros((EPAD - E, DE), jnp.float32)], axis=0)
    pospad = jnp.concatenate(
        [pos, jnp.zeros((N, PW - 3), jnp.float32)], axis=1)
    zrows = jnp.zeros((N, VW), jnp.float32)

    w = {}
    for c in range(2):
        we1 = params[f'We1_{c}']
        w[f'dst{c}'] = we1[:D]
        w[f'src{c}'] = we1[D:2 * D]
        w[f'd2_{c}'] = _r8(we1[2 * D])
        w[f'ea{c}'] = we1[2 * D + 1:]
        w[f'be1_{c}'] = _r8(params[f'be1_{c}'])
        w[f'we2_{c}'] = params[f'We2_{c}']
        w[f'be2_{c}'] = _r8(params[f'be2_{c}'])
        w[f'wp{c}'] = _r8(params[f'Wp_{c}'][:, 0])
        wh1 = params[f'Wh1_{c}']
        w[f'h1h{c}'] = wh1[:D]
        w[f'h1a{c}'] = wh1[D:]
        w[f'bh1_{c}'] = _r8(params[f'bh1_{c}'])
        w[f'wh2_{c}'] = params[f'Wh2_{c}']
        w[f'bh2_{c}'] = _r8(params[f'bh2_{c}'])

    xinj = pl.pallas_call(
        _prep_node_body,
        out_shape=jax.ShapeDtypeStruct((N, D), jnp.float32),
        compiler_params=_TC_PARAMS,
    )(x, params['Winj'], _r8(params['binj']), _r8(params['g_inj_in']),
      _r8(params['b_inj_in']), _r8(params['g_inj_out']),
      _r8(params['b_inj_out']))

    pa0, pb0 = _gather_pos(pospad, pospad, dstp, srcp)
    eaw0, eaw1, rel0 = _prep_edge_call(
        pa0, pb0, eap, w['ea0'], w['be1_0'], w['d2_0'], w['ea1'], w['be1_1'])

    g3, b3 = _r8(params['g3']), _r8(params['b3'])
    g4, b4 = _r8(params['g4']), _r8(params['b4'])
    pg1, pb1 = _r8(_pad16(params['pg1'])), _r8(_pad16(params['pb1']))
    pg2, pb2 = _r8(_pad16(params['pg2'])), _r8(_pad16(params['pb2']))
    pg1 = pg1.at[:, 3:].set(1.0)
    pg2 = pg2.at[:, 3:].set(1.0)

    z = jnp.zeros((N, D), jnp.float32)
    q = jnp.zeros((N, PW), jnp.float32)
    ta0 = tb0 = None

    node0_shapes = [
        jax.ShapeDtypeStruct((N, D), jnp.float32),
        jax.ShapeDtypeStruct((N, D), jnp.float32),
        jax.ShapeDtypeStruct((N, PW), jnp.float32),
        jax.ShapeDtypeStruct((N, DEH), jnp.float32),
        jax.ShapeDtypeStruct((N, DEH), jnp.float32),
        jax.ShapeDtypeStruct((N, PW), jnp.float32),
    ]
    node1_shapes = [
        jax.ShapeDtypeStruct((N, D), jnp.float32),
        jax.ShapeDtypeStruct((N, PW), jnp.float32),
        jax.ShapeDtypeStruct((N, DEH), jnp.float32),
        jax.ShapeDtypeStruct((N, DEH), jnp.float32),
    ]

    for it in range(NIT):
        if it == 0:
            vals0 = _edge0_first_call(eaw0, rel0, w['we2_0'], w['be2_0'],
                                      w['wp0'])
        else:
            a0, b0 = _gather_feat(ta0, tb0, dstp, srcp)
            vals0 = _edge0_call(a0, b0, eaw0, rel0, w['we2_0'], w['be2_0'],
                                w['wp0'])
        accs0 = _scatter(vals0, dst2d, zrows)
        z0, z1g, p0, ta1, tb1, p1 = pl.pallas_call(
            _node0_body, out_shape=node0_shapes, compiler_params=_TC_PARAMS,
        )(z, accs0, xinj, q, w['h1h0'], w['h1a0'], w['bh1_0'], w['wh2_0'],
          w['bh2_0'], g3, b3, pg1, pb1, w['dst1'], w['src1'])

        a1, b1, pa1, pb1_g = _gather_feat_pos(ta1, tb1, p1, p1, dstp, srcp)
        vals1 = _edge1_call(a1, b1, pa1, pb1_g, eaw1, w['d2_1'], w['we2_1'],
                            w['be2_1'], w['wp1'])
        accs1 = _scatter(vals1, dst2d, zrows)
        z, q, ta0, tb0 = pl.pallas_call(
            _node1_body, out_shape=node1_shapes, compiler_params=_TC_PARAMS,
        )(z0, z1g, accs1, p0, w['h1h1'], w['h1a1'], w['bh1_1'], w['wh2_1'],
          w['bh2_1'], g4, b4, pg2, pb2, w['dst0'], w['src0'])

    return z, q[:, :3]


# speedup vs baseline: 2.7792x; 1.1926x over previous
import functools

import jax
import jax.numpy as jnp
from jax import lax
from jax.experimental import pallas as pl
from jax.experimental.pallas import tpu as pltpu
from jax.experimental.pallas import tpu_sc as plsc

N = 10000
E = 160000
D = 128
DE = 16
DEH = 64
DM = 32
NIT = 3

NC, NS = 2, 16
NW = NC * NS
EPAD = 163840
EPW = EPAD // NW
CH = 128
NCHUNK = EPW // CH
PW = 16
VW = DM + PW
NPT = N // NS
EBLK = 2048
EPS = 1e-5


KSUP = 2
NB = 2
SUP = KSUP * CH
NSUP = NCHUNK // KSUP


def _make_gather(specs):
    n = len(specs)
    mesh = plsc.VectorSubcoreMesh(core_axis_name="c", subcore_axis_name="s")
    out_type = [jax.ShapeDtypeStruct((EPAD, w), jnp.float32) for w, _ in specs]
    scratch = [pltpu.VMEM((NCHUNK, CH), jnp.int32),
               pltpu.VMEM((NCHUNK, CH), jnp.int32)]
    scratch += [pltpu.VMEM((SUP, w), jnp.float32)
                for w, _ in specs for _b in range(NB)]
    scratch += [pltpu.SemaphoreType.DMA for _ in range(2 * n * NB)]

    @functools.partial(pl.kernel, out_type=out_type, mesh=mesh,
                       scratch_types=scratch,
                       compiler_params=pltpu.CompilerParams(
                           use_tc_tiling_on_sc=False))
    def k(*refs):
        tables = refs[:n]
        dst2d, src2d = refs[n], refs[n + 1]
        outs = refs[n + 2:2 * n + 2]
        idxd, idxs = refs[2 * n + 2], refs[2 * n + 3]
        i0 = 2 * n + 4
        rows = [[refs[i0 + t * NB + b] for b in range(NB)] for t in range(n)]
        i0 += n * NB
        gsem = [[refs[i0 + t * NB + b] for b in range(NB)] for t in range(n)]
        i0 += n * NB
        wsem = [[refs[i0 + t * NB + b] for b in range(NB)] for t in range(n)]
        wid = lax.axis_index("s") * NC + lax.axis_index("c")
        base = pl.multiple_of(wid * EPW, EPW)
        crow = pl.multiple_of(wid * NCHUNK, NCHUNK)
        pltpu.sync_copy(dst2d.at[pl.ds(crow, NCHUNK)], idxd)
        pltpu.sync_copy(src2d.at[pl.ds(crow, NCHUNK)], idxs)

        def fire(b, sc):
            for t, (_, use_src) in enumerate(specs):
                idx = idxs if use_src else idxd
                for kk in range(KSUP):
                    pltpu.async_copy(tables[t].at[idx.at[sc * KSUP + kk]],
                                     rows[t][b].at[pl.ds(kk * CH, CH)],
                                     gsem[t][b])

        def wait_fire(b):
            for t in range(n):
                for kk in range(KSUP):
                    pltpu.make_async_copy(
                        tables[t].at[idxd.at[0]],
                        rows[t][b].at[pl.ds(kk * CH, CH)],
                        gsem[t][b]).wait()

        def wb(b, sc):
            off = pl.multiple_of(base + sc * SUP, CH)
            for t in range(n):
                pltpu.async_copy(rows[t][b], outs[t].at[pl.ds(off, SUP)],
                                 wsem[t][b])

        def wait_wb(b):
            for t in range(n):
                pltpu.make_async_copy(rows[t][b],
                                      outs[t].at[pl.ds(0, SUP)],
                                      wsem[t][b]).wait()

        fire(0, 0)

        def body(i, carry):
            jj = i * 2

            @pl.when(i >= 1)
            def _drain1():
                wait_wb(1)

            fire(1, jj + 1)
            wait_fire(0)
            wb(0, jj)

            @pl.when(jj + 2 < NSUP)
            def _refill0():
                wait_wb(0)
                fire(0, jj + 2)

            wait_fire(1)
            wb(1, jj + 1)
            return carry

        lax.fori_loop(0, NSUP // 2, body, 0)
        wait_wb(0)
        wait_wb(1)

    return k


def _make_scatter():
    mesh = plsc.VectorSubcoreMesh(core_axis_name="c", subcore_axis_name="s")

    @functools.partial(
        pl.kernel,
        out_type=jax.ShapeDtypeStruct((NC, N, VW), jnp.float32),
        mesh=mesh,
        scratch_types=[
            pltpu.VMEM((NCHUNK, CH), jnp.int32),
            pltpu.VMEM((SUP, VW), jnp.float32),
            pltpu.VMEM((SUP, VW), jnp.float32),
            pltpu.VMEM_SHARED((N, VW), jnp.float32),
            pltpu.SemaphoreType.DMA,
            pltpu.SemaphoreType.DMA,
        ],
        compiler_params=pltpu.CompilerParams(use_tc_tiling_on_sc=False),
    )
    def k(vals, dst2d, zrows, out, idxv, valv0, valv1, acc, sem0, sem1):
        c = lax.axis_index("c")
        s = lax.axis_index("s")
        wid = s * NC + c
        valv = (valv0, valv1)
        sems = (sem0, sem1)

        @pl.when(s == 0)
        def _zero():
            pltpu.sync_copy(zrows, acc)

        base = pl.multiple_of(wid * EPW, EPW)
        pltpu.sync_copy(dst2d.at[pl.ds(wid * NCHUNK, NCHUNK)], idxv)
        plsc.subcore_barrier()

        def load(b, sc):
            off = pl.multiple_of(base + sc * SUP, CH)
            pltpu.async_copy(vals.at[pl.ds(off, SUP)], valv[b], sems[b])

        def wait_load(b):
            pltpu.make_async_copy(vals.at[pl.ds(0, SUP)], valv[b],
                                  sems[b]).wait()

        def scat(b, sc):
            for kk in range(KSUP):
                pltpu.sync_copy(valv[b].at[pl.ds(kk * CH, CH)],
                                acc.at[idxv.at[sc * KSUP + kk]], add=True)

        load(0, 0)

        def body(i, carry):
            jj = i * 2
            load(1, jj + 1)
            wait_load(0)
            scat(0, jj)

            @pl.when(jj + 2 < NSUP)
            def _refill0():
                load(0, jj + 2)

            wait_load(1)
            scat(1, jj + 1)
            return carry

        lax.fori_loop(0, NSUP // 2, body, 0)
        plsc.subcore_barrier()
        r0 = pl.multiple_of(s * NPT, NPT)
        pltpu.sync_copy(acc.at[pl.ds(r0, NPT)], out.at[c, pl.ds(r0, NPT)])

    return k


TW = DEH + PW

_gather_pos = _make_gather(((PW, False), (PW, True)))
_gather_feat = _make_gather(((DEH, False), (DEH, True)))
_gather_feat_pos = _make_gather(((TW, False), (TW, True)))
_scatter = _make_scatter()


def _gelu(v):
    return jax.nn.gelu(v, approximate=True)


def _ln(v, g, b):
    m = jnp.mean(v, axis=-1, keepdims=True)
    c = v - m
    s = jnp.mean(c * c, axis=-1, keepdims=True)
    return c / jnp.sqrt(s + EPS) * g + b


def _bn(v, g, b):
    m = jnp.mean(v, axis=0, keepdims=True)
    c = v - m
    s = jnp.mean(c * c, axis=0, keepdims=True)
    return c / jnp.sqrt(s + EPS) * g + b


def _dot(a, b):
    return jnp.dot(a, b, preferred_element_type=jnp.float32)


def _prep_node_body(x_ref, winj_ref, binj_ref, gi_ref, bi_ref, go_ref, bo_ref,
                    xinj_ref):
    v = _ln(x_ref[...], gi_ref[0:1], bi_ref[0:1])
    v = _dot(v, winj_ref[...]) + binj_ref[0:1]
    xinj_ref[...] = _ln(v, go_ref[0:1], bo_ref[0:1])


def _prep_edge_body(pa_ref, pb_ref, ea_ref, wea0_ref, be10_ref, wd20_ref,
                    wea1_ref, be11_ref, eaw0_ref, eaw1_ref, rel0_ref):
    rel = pb_ref[...] - pa_ref[...]
    d2 = jnp.sum(rel * rel, axis=1, keepdims=True)
    eaw0_ref[...] = _dot(ea_ref[...], wea0_ref[...]) + be10_ref[0:1] \
        + d2 * wd20_ref[0:1]
    eaw1_ref[...] = _dot(ea_ref[...], wea1_ref[...]) + be11_ref[0:1]
    rel0_ref[...] = rel


def _edge_tail(mpre, rel, we2_ref, be2_ref, wp_ref, vals_ref):
    m = _gelu(mpre)
    m2 = _gelu(_dot(m, we2_ref[...]) + be2_ref[0:1])
    w = jnp.sum(m2 * wp_ref[0:1], axis=1, keepdims=True)
    rows = pl.program_id(0) * EBLK + lax.broadcasted_iota(jnp.int32, (EBLK, 1), 0)
    mask = (rows < E).astype(jnp.float32)
    vals_ref[...] = jnp.concatenate([m2, rel * w], axis=1) * mask


def _edge0_first_body(eaw_ref, rel_ref, we2_ref, be2_ref, wp_ref, vals_ref):
    _edge_tail(eaw_ref[...], rel_ref[...], we2_ref, be2_ref, wp_ref, vals_ref)


def _edge0_body(a_ref, b_ref, eaw_ref, rel_ref, we2_ref, be2_ref, wp_ref,
                vals_ref):
    _edge_tail(a_ref[...] + b_ref[...] + eaw_ref[...], rel_ref[...],
               we2_ref, be2_ref, wp_ref, vals_ref)


def _edge1_body(a_ref, b_ref, eaw_ref, wd2_ref, we2_ref,
                be2_ref, wp_ref, vals_ref):
    a = a_ref[...]
    b = b_ref[...]
    rel = b[:, DEH:] - a[:, DEH:]
    d2 = jnp.sum(rel * rel, axis=1, keepdims=True)
    mpre = a[:, :DEH] + b[:, :DEH] + eaw_ref[...] + d2 * wd2_ref[0:1]
    _edge_tail(mpre, rel, we2_ref, be2_ref, wp_ref, vals_ref)


def _node0_body(z_ref, accs_ref, xinj_ref, q_ref, wh1h_ref, wh1a_ref, bh1_ref,
                wh2_ref, bh2_ref, g3_ref, b3_ref, pg1_ref, pb1_ref,
                z0_ref, z1g_ref, p0_ref, p1_ref):
    acc = accs_ref[0] + accs_ref[1]
    agg = acc[:, :DM]
    p0 = acc[:, DM:]
    hin = _dot(z_ref[...], wh1h_ref[...]) + _dot(agg, wh1a_ref[...]) + bh1_ref[0:1]
    z0 = _dot(_gelu(hin), wh2_ref[...]) + bh2_ref[0:1]
    z1g = _gelu(_ln(z0 + xinj_ref[...], g3_ref[0:1], b3_ref[0:1]))
    p1 = _bn(p0 + q_ref[...], pg1_ref[0:1], pb1_ref[0:1])
    z0_ref[...] = z0
    z1g_ref[...] = z1g
    p0_ref[...] = p0
    p1_ref[...] = p1


def _node1_body(z0_ref, z1g_ref, accs_ref, p0_ref, wh1h_ref, wh1a_ref,
                bh1_ref, wh2_ref, bh2_ref, g4_ref, b4_ref, pg2_ref, pb2_ref,
                z_ref, q_out_ref):
    acc = accs_ref[0] + accs_ref[1]
    agg = acc[:, :DM]
    p2 = acc[:, DM:]
    hin = _dot(z1g_ref[...], wh1h_ref[...]) + _dot(agg, wh1a_ref[...]) + bh1_ref[0:1]
    z2 = _dot(_gelu(hin), wh2_ref[...]) + bh2_ref[0:1]
    zn = _ln(_gelu(z0_ref[...] + z2), g4_ref[0:1], b4_ref[0:1])
    qn = _bn(p0_ref[...] + p2, pg2_ref[0:1], pb2_ref[0:1])
    z_ref[...] = zn
    q_out_ref[...] = qn


def _tables1_body(z1g_ref, p1_ref, wdst_ref, wsrc_ref, ta_ref, tb_ref):
    z1g = z1g_ref[...]
    p1 = p1_ref[...]
    ta_ref[...] = jnp.concatenate([_dot(z1g, wdst_ref[...]), p1], axis=1)
    tb_ref[...] = jnp.concatenate([_dot(z1g, wsrc_ref[...]), p1], axis=1)


def _tables0_body(z_ref, wdst_ref, wsrc_ref, ta_ref, tb_ref):
    z = z_ref[...]
    ta_ref[...] = _dot(z, wdst_ref[...])
    tb_ref[...] = _dot(z, wsrc_ref[...])


_TC_PARAMS = pltpu.CompilerParams(vmem_limit_bytes=100 * 1024 * 1024)


def _full(shape):
    return pl.BlockSpec(shape, lambda i: (0, 0))


def _eblk(width):
    return pl.BlockSpec((EBLK, width), lambda i: (i, 0))


_EGRID = EPAD // EBLK


def _prep_edge_call(pa, pb, ea, wea0, be10, wd20, wea1, be11):
    return pl.pallas_call(
        _prep_edge_body,
        grid=(_EGRID,),
        in_specs=[_eblk(PW), _eblk(PW), _eblk(DE), _full((DE, DEH)),
                  _full((8, DEH)), _full((8, DEH)), _full((DE, DEH)),
                  _full((8, DEH))],
        out_specs=[_eblk(DEH), _eblk(DEH), _eblk(PW)],
        out_shape=[jax.ShapeDtypeStruct((EPAD, DEH), jnp.float32),
                   jax.ShapeDtypeStruct((EPAD, DEH), jnp.float32),
                   jax.ShapeDtypeStruct((EPAD, PW), jnp.float32)],
    )(pa, pb, ea, wea0, be10, wd20, wea1, be11)


def _edge0_first_call(eaw, rel0, we2, be2, wp):
    return pl.pallas_call(
        _edge0_first_body,
        grid=(_EGRID,),
        in_specs=[_eblk(DEH), _eblk(PW), _full((DEH, DM)), _full((8, DM)),
                  _full((8, DM))],
        out_specs=_eblk(VW),
        out_shape=jax.ShapeDtypeStruct((EPAD, VW), jnp.float32),
    )(eaw, rel0, we2, be2, wp)


def _edge0_call(a, b, eaw, rel0, we2, be2, wp):
    return pl.pallas_call(
        _edge0_body,
        grid=(_EGRID,),
        in_specs=[_eblk(DEH), _eblk(DEH), _eblk(DEH), _eblk(PW),
                  _full((DEH, DM)), _full((8, DM)), _full((8, DM))],
        out_specs=_eblk(VW),
        out_shape=jax.ShapeDtypeStruct((EPAD, VW), jnp.float32),
    )(a, b, eaw, rel0, we2, be2, wp)


def _edge1_call(a, b, eaw, wd2, we2, be2, wp):
    return pl.pallas_call(
        _edge1_body,
        grid=(_EGRID,),
        in_specs=[_eblk(TW), _eblk(TW), _eblk(DEH),
                  _full((8, DEH)), _full((DEH, DM)), _full((8, DM)),
                  _full((8, DM))],
        out_specs=_eblk(VW),
        out_shape=jax.ShapeDtypeStruct((EPAD, VW), jnp.float32),
    )(a, b, eaw, wd2, we2, be2, wp)


def _r8(v):
    return jnp.broadcast_to(v.astype(jnp.float32)[None, :], (8, v.shape[0]))


def _pad16(v):
    return jnp.concatenate([v, jnp.zeros((PW - v.shape[0],), v.dtype)])


def kernel(x, pos, edge_index, edge_attr, params):
    src = edge_index[0].astype(jnp.int32)
    dst = edge_index[1].astype(jnp.int32)
    zpad = jnp.zeros((EPAD - E,), jnp.int32)
    dstp = jnp.concatenate([dst, zpad])
    srcp = jnp.concatenate([src, zpad])
    dst2d = dstp.reshape(EPAD // CH, CH)
    src2d = srcp.reshape(EPAD // CH, CH)
    eap = jnp.concatenate(
        [edge_attr, jnp.zeros((EPAD - E, DE), jnp.float32)], axis=0)
    pospad = jnp.concatenate(
        [pos, jnp.zeros((N, PW - 3), jnp.float32)], axis=1)
    zrows = jnp.zeros((N, VW), jnp.float32)

    w = {}
    for c in range(2):
        we1 = params[f'We1_{c}']
        w[f'dst{c}'] = we1[:D]
        w[f'src{c}'] = we1[D:2 * D]
        w[f'd2_{c}'] = _r8(we1[2 * D])
        w[f'ea{c}'] = we1[2 * D + 1:]
        w[f'be1_{c}'] = _r8(params[f'be1_{c}'])
        w[f'we2_{c}'] = params[f'We2_{c}']
        w[f'be2_{c}'] = _r8(params[f'be2_{c}'])
        w[f'wp{c}'] = _r8(params[f'Wp_{c}'][:, 0])
        wh1 = params[f'Wh1_{c}']
        w[f'h1h{c}'] = wh1[:D]
        w[f'h1a{c}'] = wh1[D:]
        w[f'bh1_{c}'] = _r8(params[f'bh1_{c}'])
        w[f'wh2_{c}'] = params[f'Wh2_{c}']
        w[f'bh2_{c}'] = _r8(params[f'bh2_{c}'])

    xinj = pl.pallas_call(
        _prep_node_body,
        out_shape=jax.ShapeDtypeStruct((N, D), jnp.float32),
        compiler_params=_TC_PARAMS,
    )(x, params['Winj'], _r8(params['binj']), _r8(params['g_inj_in']),
      _r8(params['b_inj_in']), _r8(params['g_inj_out']),
      _r8(params['b_inj_out']))

    pa0, pb0 = _gather_pos(pospad, pospad, dst2d, src2d)
    eaw0, eaw1, rel0 = _prep_edge_call(
        pa0, pb0, eap, w['ea0'], w['be1_0'], w['d2_0'], w['ea1'], w['be1_1'])

    g3, b3 = _r8(params['g3']), _r8(params['b3'])
    g4, b4 = _r8(params['g4']), _r8(params['b4'])
    pg1, pb1 = _r8(_pad16(params['pg1'])), _r8(_pad16(params['pb1']))
    pg2, pb2 = _r8(_pad16(params['pg2'])), _r8(_pad16(params['pb2']))
    pg1 = pg1.at[:, 3:].set(1.0)
    pg2 = pg2.at[:, 3:].set(1.0)

    z = jnp.zeros((N, D), jnp.float32)
    q = jnp.zeros((N, PW), jnp.float32)
    ta0 = tb0 = None

    node0_shapes = [
        jax.ShapeDtypeStruct((N, D), jnp.float32),
        jax.ShapeDtypeStruct((N, D), jnp.float32),
        jax.ShapeDtypeStruct((N, PW), jnp.float32),
        jax.ShapeDtypeStruct((N, PW), jnp.float32),
    ]
    node1_shapes = [
        jax.ShapeDtypeStruct((N, D), jnp.float32),
        jax.ShapeDtypeStruct((N, PW), jnp.float32),
    ]
    tables1_shapes = [
        jax.ShapeDtypeStruct((N, TW), jnp.float32),
        jax.ShapeDtypeStruct((N, TW), jnp.float32),
    ]
    tables0_shapes = [
        jax.ShapeDtypeStruct((N, DEH), jnp.float32),
        jax.ShapeDtypeStruct((N, DEH), jnp.float32),
    ]

    for it in range(NIT):
        if it == 0:
            vals0 = _edge0_first_call(eaw0, rel0, w['we2_0'], w['be2_0'],
                                      w['wp0'])
        else:
            a0, b0 = _gather_feat(ta0, tb0, dst2d, src2d)
            vals0 = _edge0_call(a0, b0, eaw0, rel0, w['we2_0'], w['be2_0'],
                                w['wp0'])
        accs0 = _scatter(vals0, dst2d, zrows)
        z0, z1g, p0, p1 = pl.pallas_call(
            _node0_body, out_shape=node0_shapes, compiler_params=_TC_PARAMS,
        )(z, accs0, xinj, q, w['h1h0'], w['h1a0'], w['bh1_0'], w['wh2_0'],
          w['bh2_0'], g3, b3, pg1, pb1)
        ta1, tb1 = pl.pallas_call(
            _tables1_body, out_shape=tables1_shapes, compiler_params=_TC_PARAMS,
        )(z1g, p1, w['dst1'], w['src1'])

        a1, b1 = _gather_feat_pos(ta1, tb1, dst2d, src2d)
        vals1 = _edge1_call(a1, b1, eaw1, w['d2_1'], w['we2_1'],
                            w['be2_1'], w['wp1'])
        accs1 = _scatter(vals1, dst2d, zrows)
        z, q = pl.pallas_call(
            _node1_body, out_shape=node1_shapes, compiler_params=_TC_PARAMS,
        )(z0, z1g, accs1, p0, w['h1h1'], w['h1a1'], w['bh1_1'], w['wh2_1'],
          w['bh2_1'], g4, b4, pg2, pb2)
        if it + 1 < NIT:
            ta0, tb0 = pl.pallas_call(
                _tables0_body, out_shape=tables0_shapes,
                compiler_params=_TC_PARAMS,
            )(z, w['dst0'], w['src0'])

    return z, q[:, :3]


# speedup vs baseline: 2.8451x; 1.0237x over previous
import functools

import jax
import jax.numpy as jnp
from jax import lax
from jax.experimental import pallas as pl
from jax.experimental.pallas import tpu as pltpu
from jax.experimental.pallas import tpu_sc as plsc

N = 10000
E = 160000
D = 128
DE = 16
DEH = 64
DM = 32
NIT = 3

NC, NS = 2, 16
NW = NC * NS
EPAD = 163840
EPW = EPAD // NW
CH = 128
NCHUNK = EPW // CH
PW = 16
VW = DM + PW
NPT = N // NS
EBLK = 2048
EPS = 1e-5


KSUP = 2
NB = 2
SUP = KSUP * CH
NSUP = NCHUNK // KSUP


def _make_gather(specs):
    n = len(specs)
    mesh = plsc.VectorSubcoreMesh(core_axis_name="c", subcore_axis_name="s")
    out_type = [jax.ShapeDtypeStruct((EPAD, w), jnp.float32) for w, _ in specs]
    scratch = [pltpu.VMEM((NCHUNK, CH), jnp.int32),
               pltpu.VMEM((NCHUNK, CH), jnp.int32)]
    scratch += [pltpu.VMEM((SUP, w), jnp.float32)
                for w, _ in specs for _b in range(NB)]
    scratch += [pltpu.SemaphoreType.DMA for _ in range(2 * n * NB)]

    @functools.partial(pl.kernel, out_type=out_type, mesh=mesh,
                       scratch_types=scratch,
                       compiler_params=pltpu.CompilerParams(
                           use_tc_tiling_on_sc=False))
    def k(*refs):
        tables = refs[:n]
        dst2d, src2d = refs[n], refs[n + 1]
        outs = refs[n + 2:2 * n + 2]
        idxd, idxs = refs[2 * n + 2], refs[2 * n + 3]
        i0 = 2 * n + 4
        rows = [[refs[i0 + t * NB + b] for b in range(NB)] for t in range(n)]
        i0 += n * NB
        gsem = [[refs[i0 + t * NB + b] for b in range(NB)] for t in range(n)]
        i0 += n * NB
        wsem = [[refs[i0 + t * NB + b] for b in range(NB)] for t in range(n)]
        wid = lax.axis_index("s") * NC + lax.axis_index("c")
        base = pl.multiple_of(wid * EPW, EPW)
        crow = pl.multiple_of(wid * NCHUNK, NCHUNK)
        pltpu.sync_copy(dst2d.at[pl.ds(crow, NCHUNK)], idxd)
        pltpu.sync_copy(src2d.at[pl.ds(crow, NCHUNK)], idxs)

        def fire(b, sc):
            for t, (_, use_src) in enumerate(specs):
                idx = idxs if use_src else idxd
                for kk in range(KSUP):
                    pltpu.async_copy(tables[t].at[idx.at[sc * KSUP + kk]],
                                     rows[t][b].at[pl.ds(kk * CH, CH)],
                                     gsem[t][b])

        def wait_fire(b):
            for t in range(n):
                for kk in range(KSUP):
                    pltpu.make_async_copy(
                        tables[t].at[idxd.at[0]],
                        rows[t][b].at[pl.ds(kk * CH, CH)],
                        gsem[t][b]).wait()

        def wb(b, sc):
            off = pl.multiple_of(base + sc * SUP, CH)
            for t in range(n):
                pltpu.async_copy(rows[t][b], outs[t].at[pl.ds(off, SUP)],
                                 wsem[t][b])

        def wait_wb(b):
            for t in range(n):
                pltpu.make_async_copy(rows[t][b],
                                      outs[t].at[pl.ds(0, SUP)],
                                      wsem[t][b]).wait()

        fire(0, 0)

        def body(i, carry):
            jj = i * 2

            @pl.when(i >= 1)
            def _drain1():
                wait_wb(1)

            fire(1, jj + 1)
            wait_fire(0)
            wb(0, jj)

            @pl.when(jj + 2 < NSUP)
            def _refill0():
                wait_wb(0)
                fire(0, jj + 2)

            wait_fire(1)
            wb(1, jj + 1)
            return carry

        lax.fori_loop(0, NSUP // 2, body, 0)
        wait_wb(0)
        wait_wb(1)

    return k


def _make_scatter():
    mesh = plsc.VectorSubcoreMesh(core_axis_name="c", subcore_axis_name="s")

    @functools.partial(
        pl.kernel,
        out_type=jax.ShapeDtypeStruct((NC, N, VW), jnp.float32),
        mesh=mesh,
        scratch_types=[
            pltpu.VMEM((NCHUNK, CH), jnp.int32),
            pltpu.VMEM((SUP, VW), jnp.float32),
            pltpu.VMEM((SUP, VW), jnp.float32),
            pltpu.VMEM_SHARED((N, VW), jnp.float32),
            pltpu.SemaphoreType.DMA,
            pltpu.SemaphoreType.DMA,
            pltpu.SemaphoreType.DMA,
            pltpu.SemaphoreType.DMA,
        ],
        compiler_params=pltpu.CompilerParams(use_tc_tiling_on_sc=False),
    )
    def k(vals, dst2d, zrows, out, idxv, valv0, valv1, acc, sem0, sem1,
          asem0, asem1):
        c = lax.axis_index("c")
        s = lax.axis_index("s")
        wid = s * NC + c
        valv = (valv0, valv1)
        sems = (sem0, sem1)
        asems = (asem0, asem1)

        @pl.when(s == 0)
        def _zero():
            pltpu.sync_copy(zrows, acc)

        base = pl.multiple_of(wid * EPW, EPW)
        pltpu.sync_copy(dst2d.at[pl.ds(wid * NCHUNK, NCHUNK)], idxv)
        plsc.subcore_barrier()

        def load(b, sc):
            off = pl.multiple_of(base + sc * SUP, CH)
            pltpu.async_copy(vals.at[pl.ds(off, SUP)], valv[b], sems[b])

        def wait_load(b):
            pltpu.make_async_copy(vals.at[pl.ds(0, SUP)], valv[b],
                                  sems[b]).wait()

        def scat(b, sc):
            for kk in range(KSUP):
                pltpu.async_copy(valv[b].at[pl.ds(kk * CH, CH)],
                                 acc.at[idxv.at[sc * KSUP + kk]], asems[b],
                                 add=True)

        def wait_scat(b):
            for kk in range(KSUP):
                pltpu.make_async_copy(valv[b].at[pl.ds(kk * CH, CH)],
                                      acc.at[idxv.at[0]], asems[b]).wait()

        load(0, 0)

        def body(i, carry):
            jj = i * 2

            @pl.when(i >= 1)
            def _drain1():
                wait_scat(1)

            load(1, jj + 1)
            wait_load(0)
            scat(0, jj)

            @pl.when(jj + 2 < NSUP)
            def _refill0():
                wait_scat(0)
                load(0, jj + 2)

            wait_load(1)
            scat(1, jj + 1)
            return carry

        lax.fori_loop(0, NSUP // 2, body, 0)
        wait_scat(0)
        wait_scat(1)
        plsc.subcore_barrier()
        r0 = pl.multiple_of(s * NPT, NPT)
        pltpu.sync_copy(acc.at[pl.ds(r0, NPT)], out.at[c, pl.ds(r0, NPT)])

    return k


TW = DEH + PW

_gather_pos = _make_gather(((PW, False), (PW, True)))
_gather_feat = _make_gather(((DEH, False), (DEH, True)))
_gather_feat_pos = _make_gather(((TW, False), (TW, True)))
_scatter = _make_scatter()


def _gelu(v):
    return jax.nn.gelu(v, approximate=True)


def _ln(v, g, b):
    m = jnp.mean(v, axis=-1, keepdims=True)
    c = v - m
    s = jnp.mean(c * c, axis=-1, keepdims=True)
    return c / jnp.sqrt(s + EPS) * g + b


def _bn(v, g, b):
    m = jnp.mean(v, axis=0, keepdims=True)
    c = v - m
    s = jnp.mean(c * c, axis=0, keepdims=True)
    return c / jnp.sqrt(s + EPS) * g + b


def _dot(a, b):
    return jnp.dot(a, b, preferred_element_type=jnp.float32)


def _prep_node_body(x_ref, winj_ref, binj_ref, gi_ref, bi_ref, go_ref, bo_ref,
                    xinj_ref):
    v = _ln(x_ref[...], gi_ref[0:1], bi_ref[0:1])
    v = _dot(v, winj_ref[...]) + binj_ref[0:1]
    xinj_ref[...] = _ln(v, go_ref[0:1], bo_ref[0:1])


def _prep_edge_body(pa_ref, pb_ref, ea_ref, wea0_ref, be10_ref, wd20_ref,
                    wea1_ref, be11_ref, eaw0_ref, eaw1_ref, rel0_ref):
    rel = pb_ref[...] - pa_ref[...]
    d2 = jnp.sum(rel * rel, axis=1, keepdims=True)
    eaw0_ref[...] = _dot(ea_ref[...], wea0_ref[...]) + be10_ref[0:1] \
        + d2 * wd20_ref[0:1]
    eaw1_ref[...] = _dot(ea_ref[...], wea1_ref[...]) + be11_ref[0:1]
    rel0_ref[...] = rel


def _edge_tail(mpre, rel, we2_ref, be2_ref, wp_ref, vals_ref):
    m = _gelu(mpre)
    m2 = _gelu(_dot(m, we2_ref[...]) + be2_ref[0:1])
    w = jnp.sum(m2 * wp_ref[0:1], axis=1, keepdims=True)
    rows = pl.program_id(0) * EBLK + lax.broadcasted_iota(jnp.int32, (EBLK, 1), 0)
    mask = (rows < E).astype(jnp.float32)
    vals_ref[...] = jnp.concatenate([m2, rel * w], axis=1) * mask


def _edge0_first_body(eaw_ref, rel_ref, we2_ref, be2_ref, wp_ref, vals_ref):
    _edge_tail(eaw_ref[...], rel_ref[...], we2_ref, be2_ref, wp_ref, vals_ref)


def _edge0_body(a_ref, b_ref, eaw_ref, rel_ref, we2_ref, be2_ref, wp_ref,
                vals_ref):
    _edge_tail(a_ref[...] + b_ref[...] + eaw_ref[...], rel_ref[...],
               we2_ref, be2_ref, wp_ref, vals_ref)


def _edge1_body(a_ref, b_ref, eaw_ref, wd2_ref, we2_ref,
                be2_ref, wp_ref, vals_ref):
    a = a_ref[...]
    b = b_ref[...]
    rel = b[:, DEH:] - a[:, DEH:]
    d2 = jnp.sum(rel * rel, axis=1, keepdims=True)
    mpre = a[:, :DEH] + b[:, :DEH] + eaw_ref[...] + d2 * wd2_ref[0:1]
    _edge_tail(mpre, rel, we2_ref, be2_ref, wp_ref, vals_ref)


def _node0_body(z_ref, accs_ref, xinj_ref, q_ref, wh1h_ref, wh1a_ref, bh1_ref,
                wh2_ref, bh2_ref, g3_ref, b3_ref, pg1_ref, pb1_ref,
                z0_ref, z1g_ref, p0_ref, p1_ref):
    acc = accs_ref[0] + accs_ref[1]
    agg = acc[:, :DM]
    p0 = acc[:, DM:]
    hin = _dot(z_ref[...], wh1h_ref[...]) + _dot(agg, wh1a_ref[...]) + bh1_ref[0:1]
    z0 = _dot(_gelu(hin), wh2_ref[...]) + bh2_ref[0:1]
    z1g = _gelu(_ln(z0 + xinj_ref[...], g3_ref[0:1], b3_ref[0:1]))
    p1 = _bn(p0 + q_ref[...], pg1_ref[0:1], pb1_ref[0:1])
    z0_ref[...] = z0
    z1g_ref[...] = z1g
    p0_ref[...] = p0
    p1_ref[...] = p1


def _node1_body(z0_ref, z1g_ref, accs_ref, p0_ref, wh1h_ref, wh1a_ref,
                bh1_ref, wh2_ref, bh2_ref, g4_ref, b4_ref, pg2_ref, pb2_ref,
                wdst0_ref, wsrc0_ref, z_ref, q_out_ref, ta0_ref, tb0_ref):
    acc = accs_ref[0] + accs_ref[1]
    agg = acc[:, :DM]
    p2 = acc[:, DM:]
    hin = _dot(z1g_ref[...], wh1h_ref[...]) + _dot(agg, wh1a_ref[...]) + bh1_ref[0:1]
    z2 = _dot(_gelu(hin), wh2_ref[...]) + bh2_ref[0:1]
    zn = _ln(_gelu(z0_ref[...] + z2), g4_ref[0:1], b4_ref[0:1])
    qn = _bn(p0_ref[...] + p2, pg2_ref[0:1], pb2_ref[0:1])
    z_ref[...] = zn
    q_out_ref[...] = qn
    ta0_ref[...] = _dot(zn, wdst0_ref[...])
    tb0_ref[...] = _dot(zn, wsrc0_ref[...])


def _tables1_body(z1g_ref, p1_ref, wdst_ref, wsrc_ref, ta_ref, tb_ref):
    z1g = z1g_ref[...]
    p1 = p1_ref[...]
    ta_ref[...] = jnp.concatenate([_dot(z1g, wdst_ref[...]), p1], axis=1)
    tb_ref[...] = jnp.concatenate([_dot(z1g, wsrc_ref[...]), p1], axis=1)


_TC_PARAMS = pltpu.CompilerParams(vmem_limit_bytes=100 * 1024 * 1024)


def _full(shape):
    return pl.BlockSpec(shape, lambda i: (0, 0))


def _eblk(width):
    return pl.BlockSpec((EBLK, width), lambda i: (i, 0))


_EGRID = EPAD // EBLK


def _prep_edge_call(pa, pb, ea, wea0, be10, wd20, wea1, be11):
    return pl.pallas_call(
        _prep_edge_body,
        grid=(_EGRID,),
        in_specs=[_eblk(PW), _eblk(PW), _eblk(DE), _full((DE, DEH)),
                  _full((8, DEH)), _full((8, DEH)), _full((DE, DEH)),
                  _full((8, DEH))],
        out_specs=[_eblk(DEH), _eblk(DEH), _eblk(PW)],
        out_shape=[jax.ShapeDtypeStruct((EPAD, DEH), jnp.float32),
                   jax.ShapeDtypeStruct((EPAD, DEH), jnp.float32),
                   jax.ShapeDtypeStruct((EPAD, PW), jnp.float32)],
    )(pa, pb, ea, wea0, be10, wd20, wea1, be11)


def _edge0_first_call(eaw, rel0, we2, be2, wp):
    return pl.pallas_call(
        _edge0_first_body,
        grid=(_EGRID,),
        in_specs=[_eblk(DEH), _eblk(PW), _full((DEH, DM)), _full((8, DM)),
                  _full((8, DM))],
        out_specs=_eblk(VW),
        out_shape=jax.ShapeDtypeStruct((EPAD, VW), jnp.float32),
    )(eaw, rel0, we2, be2, wp)


def _edge0_call(a, b, eaw, rel0, we2, be2, wp):
    return pl.pallas_call(
        _edge0_body,
        grid=(_EGRID,),
        in_specs=[_eblk(DEH), _eblk(DEH), _eblk(DEH), _eblk(PW),
                  _full((DEH, DM)), _full((8, DM)), _full((8, DM))],
        out_specs=_eblk(VW),
        out_shape=jax.ShapeDtypeStruct((EPAD, VW), jnp.float32),
    )(a, b, eaw, rel0, we2, be2, wp)


def _edge1_call(a, b, eaw, wd2, we2, be2, wp):
    return pl.pallas_call(
        _edge1_body,
        grid=(_EGRID,),
        in_specs=[_eblk(TW), _eblk(TW), _eblk(DEH),
                  _full((8, DEH)), _full((DEH, DM)), _full((8, DM)),
                  _full((8, DM))],
        out_specs=_eblk(VW),
        out_shape=jax.ShapeDtypeStruct((EPAD, VW), jnp.float32),
    )(a, b, eaw, wd2, we2, be2, wp)


def _r8(v):
    return jnp.broadcast_to(v.astype(jnp.float32)[None, :], (8, v.shape[0]))


def _pad16(v):
    return jnp.concatenate([v, jnp.zeros((PW - v.shape[0],), v.dtype)])


def kernel(x, pos, edge_index, edge_attr, params):
    src = edge_index[0].astype(jnp.int32)
    dst = edge_index[1].astype(jnp.int32)
    zpad = jnp.zeros((EPAD - E,), jnp.int32)
    dstp = jnp.concatenate([dst, zpad])
    srcp = jnp.concatenate([src, zpad])
    dst2d = dstp.reshape(EPAD // CH, CH)
    src2d = srcp.reshape(EPAD // CH, CH)
    eap = jnp.concatenate(
        [edge_attr, jnp.zeros((EPAD - E, DE), jnp.float32)], axis=0)
    pospad = jnp.concatenate(
        [pos, jnp.zeros((N, PW - 3), jnp.float32)], axis=1)
    zrows = jnp.zeros((N, VW), jnp.float32)

    w = {}
    for c in range(2):
        we1 = params[f'We1_{c}']
        w[f'dst{c}'] = we1[:D]
        w[f'src{c}'] = we1[D:2 * D]
        w[f'd2_{c}'] = _r8(we1[2 * D])
        w[f'ea{c}'] = we1[2 * D + 1:]
        w[f'be1_{c}'] = _r8(params[f'be1_{c}'])
        w[f'we2_{c}'] = params[f'We2_{c}']
        w[f'be2_{c}'] = _r8(params[f'be2_{c}'])
        w[f'wp{c}'] = _r8(params[f'Wp_{c}'][:, 0])
        wh1 = params[f'Wh1_{c}']
        w[f'h1h{c}'] = wh1[:D]
        w[f'h1a{c}'] = wh1[D:]
        w[f'bh1_{c}'] = _r8(params[f'bh1_{c}'])
        w[f'wh2_{c}'] = params[f'Wh2_{c}']
        w[f'bh2_{c}'] = _r8(params[f'bh2_{c}'])

    xinj = pl.pallas_call(
        _prep_node_body,
        out_shape=jax.ShapeDtypeStruct((N, D), jnp.float32),
        compiler_params=_TC_PARAMS,
    )(x, params['Winj'], _r8(params['binj']), _r8(params['g_inj_in']),
      _r8(params['b_inj_in']), _r8(params['g_inj_out']),
      _r8(params['b_inj_out']))

    pa0, pb0 = _gather_pos(pospad, pospad, dst2d, src2d)
    eaw0, eaw1, rel0 = _prep_edge_call(
        pa0, pb0, eap, w['ea0'], w['be1_0'], w['d2_0'], w['ea1'], w['be1_1'])

    g3, b3 = _r8(params['g3']), _r8(params['b3'])
    g4, b4 = _r8(params['g4']), _r8(params['b4'])
    pg1, pb1 = _r8(_pad16(params['pg1'])), _r8(_pad16(params['pb1']))
    pg2, pb2 = _r8(_pad16(params['pg2'])), _r8(_pad16(params['pb2']))
    pg1 = pg1.at[:, 3:].set(1.0)
    pg2 = pg2.at[:, 3:].set(1.0)

    z = jnp.zeros((N, D), jnp.float32)
    q = jnp.zeros((N, PW), jnp.float32)
    ta0 = tb0 = None

    node0_shapes = [
        jax.ShapeDtypeStruct((N, D), jnp.float32),
        jax.ShapeDtypeStruct((N, D), jnp.float32),
        jax.ShapeDtypeStruct((N, PW), jnp.float32),
        jax.ShapeDtypeStruct((N, PW), jnp.float32),
    ]
    node1_shapes = [
        jax.ShapeDtypeStruct((N, D), jnp.float32),
        jax.ShapeDtypeStruct((N, PW), jnp.float32),
        jax.ShapeDtypeStruct((N, DEH), jnp.float32),
        jax.ShapeDtypeStruct((N, DEH), jnp.float32),
    ]
    tables1_shapes = [
        jax.ShapeDtypeStruct((N, TW), jnp.float32),
        jax.ShapeDtypeStruct((N, TW), jnp.float32),
    ]

    for it in range(NIT):
        if it == 0:
            vals0 = _edge0_first_call(eaw0, rel0, w['we2_0'], w['be2_0'],
                                      w['wp0'])
        else:
            a0, b0 = _gather_feat(ta0, tb0, dst2d, src2d)
            vals0 = _edge0_call(a0, b0, eaw0, rel0, w['we2_0'], w['be2_0'],
                                w['wp0'])
        accs0 = _scatter(vals0, dst2d, zrows)
        z0, z1g, p0, p1 = pl.pallas_call(
            _node0_body, out_shape=node0_shapes, compiler_params=_TC_PARAMS,
        )(z, accs0, xinj, q, w['h1h0'], w['h1a0'], w['bh1_0'], w['wh2_0'],
          w['bh2_0'], g3, b3, pg1, pb1)
        ta1, tb1 = pl.pallas_call(
            _tables1_body, out_shape=tables1_shapes, compiler_params=_TC_PARAMS,
        )(z1g, p1, w['dst1'], w['src1'])

        a1, b1 = _gather_feat_pos(ta1, tb1, dst2d, src2d)
        vals1 = _edge1_call(a1, b1, eaw1, w['d2_1'], w['we2_1'],
                            w['be2_1'], w['wp1'])
        accs1 = _scatter(vals1, dst2d, zrows)
        z, q, ta0, tb0 = pl.pallas_call(
            _node1_body, out_shape=node1_shapes, compiler_params=_TC_PARAMS,
        )(z0, z1g, accs1, p0, w['h1h1'], w['h1a1'], w['bh1_1'], w['wh2_1'],
          w['bh2_1'], g4, b4, pg2, pb2, w['dst0'], w['src0'])

    return z, q[:, :3]
